# baseline (device time: 35740 ns/iter reference)
import jax
import jax.numpy as jnp
from jax import lax
from jax.experimental import pallas as pl
from jax.experimental.pallas import tpu as pltpu

N_ROWS = 1024
HALF = 512


def kernel(partial, gamma):
    _, m2, d = partial.shape
    gamma2 = gamma.reshape(1, d)

    def body(p_ref, g_ref, out_ref, sendx, recvx, recvy, sems):
        my_x = lax.axis_index("x")
        my_y = lax.axis_index("y")
        xpeer = (1 - my_x, my_y)
        ypeer = (my_x, 1 - my_y)

        barrier = pltpu.get_barrier_semaphore()
        for nbr in (xpeer, ypeer):
            pl.semaphore_signal(
                barrier, inc=1, device_id=nbr,
                device_id_type=pl.DeviceIdType.MESH,
            )
        pl.semaphore_wait(barrier, 2)

        peer_rows = (1 - my_x) * N_ROWS + my_y * HALF
        sendx[...] = p_ref[0, pl.ds(peer_rows, HALF), :].astype(jnp.bfloat16)
        rx = pltpu.make_async_remote_copy(
            src_ref=sendx, dst_ref=recvx,
            send_sem=sems.at[0], recv_sem=sems.at[1],
            device_id=xpeer, device_id_type=pl.DeviceIdType.MESH,
        )
        rx.start()
        rx.wait()

        ry = pltpu.make_async_remote_copy(
            src_ref=recvx, dst_ref=recvy,
            send_sem=sems.at[2], recv_sem=sems.at[3],
            device_id=ypeer, device_id_type=pl.DeviceIdType.MESH,
        )
        ry.start()
        ry.wait()

        my_rows = my_x * N_ROWS
        g = g_ref[...]

        def norm_store(half_idx, peer_bf16):
            s = p_ref[0, pl.ds(my_rows + half_idx * HALF, HALF), :] + (
                peer_bf16.astype(jnp.float32)
            )
            rms = jnp.sqrt(jnp.mean(s * s, axis=-1, keepdims=True) + 1e-6)
            out_ref[pl.ds(half_idx * HALF, HALF), :] = s / rms * g

        norm_store(my_y, recvx[...])
        norm_store(1 - my_y, recvy[...])

    return pl.pallas_call(
        body,
        out_shape=jax.ShapeDtypeStruct((N_ROWS, d), jnp.float32),
        in_specs=[
            pl.BlockSpec(memory_space=pltpu.VMEM),
            pl.BlockSpec(memory_space=pltpu.VMEM),
        ],
        out_specs=pl.BlockSpec(memory_space=pltpu.VMEM),
        scratch_shapes=[
            pltpu.VMEM((HALF, d), jnp.bfloat16),
            pltpu.VMEM((HALF, d), jnp.bfloat16),
            pltpu.VMEM((HALF, d), jnp.bfloat16),
            pltpu.SemaphoreType.DMA((4,)),
        ],
        compiler_params=pltpu.CompilerParams(collective_id=0),
    )(partial, gamma2)


# device time: 26946 ns/iter; 1.3264x vs baseline; 1.3264x over previous
import jax
import jax.numpy as jnp
from jax import lax
from jax.experimental import pallas as pl
from jax.experimental.pallas import tpu as pltpu

N_ROWS = 1024
HALF = 512
K = 4
CH = HALF // K


def kernel(partial, gamma):
    _, m2, d = partial.shape
    gamma2 = gamma.reshape(1, d)

    def body(p_ref, g_ref, out_ref, lrows, psrc, sendx, recvx, recvy,
             lsems, sx_sems, rx_sems, sy_sems, ry_sems):
        my_x = lax.axis_index("x")
        my_y = lax.axis_index("y")
        xpeer = (1 - my_x, my_y)
        ypeer = (my_x, 1 - my_y)

        peer_base = (1 - my_x) * N_ROWS + my_y * HALF
        cp_psrc = pltpu.make_async_copy(
            p_ref.at[0, pl.ds(peer_base, HALF), :], psrc, lsems.at[0])
        cp_psrc.start()
        cp_lrows = pltpu.make_async_copy(
            p_ref.at[0, pl.ds(my_x * N_ROWS, N_ROWS), :], lrows, lsems.at[1])
        cp_lrows.start()

        barrier = pltpu.get_barrier_semaphore()
        for nbr in (xpeer, ypeer):
            pl.semaphore_signal(
                barrier, inc=1, device_id=nbr,
                device_id_type=pl.DeviceIdType.MESH,
            )
        pl.semaphore_wait(barrier, 2)

        cp_psrc.wait()
        rx = []
        for k in range(K):
            sl = pl.ds(k * CH, CH)
            sendx[sl, :] = psrc[sl, :].astype(jnp.bfloat16)
            r = pltpu.make_async_remote_copy(
                src_ref=sendx.at[sl], dst_ref=recvx.at[sl],
                send_sem=sx_sems.at[k], recv_sem=rx_sems.at[k],
                device_id=xpeer, device_id_type=pl.DeviceIdType.MESH,
            )
            r.start()
            rx.append(r)

        cp_lrows.wait()
        g = g_ref[...]

        def norm_store(lrow_start, out_start, peer_bf16):
            s = lrows[pl.ds(lrow_start, CH), :] + peer_bf16.astype(jnp.float32)
            rms = jnp.sqrt(jnp.mean(s * s, axis=-1, keepdims=True) + 1e-6)
            out_ref[pl.ds(out_start, CH), :] = s / rms * g

        ry = []
        for k in range(K):
            sl = pl.ds(k * CH, CH)
            rx[k].wait_recv()
            r = pltpu.make_async_remote_copy(
                src_ref=recvx.at[sl], dst_ref=recvy.at[sl],
                send_sem=sy_sems.at[k], recv_sem=ry_sems.at[k],
                device_id=ypeer, device_id_type=pl.DeviceIdType.MESH,
            )
            r.start()
            ry.append(r)
            norm_store(my_y * HALF + k * CH, my_y * HALF + k * CH,
                       recvx[sl, :])

        for k in range(K):
            sl = pl.ds(k * CH, CH)
            ry[k].wait_recv()
            norm_store((1 - my_y) * HALF + k * CH, (1 - my_y) * HALF + k * CH,
                       recvy[sl, :])

        for k in range(K):
            rx[k].wait_send()
            ry[k].wait_send()

    return pl.pallas_call(
        body,
        out_shape=jax.ShapeDtypeStruct((N_ROWS, d), jnp.float32),
        in_specs=[
            pl.BlockSpec(memory_space=pl.ANY),
            pl.BlockSpec(memory_space=pltpu.VMEM),
        ],
        out_specs=pl.BlockSpec(memory_space=pltpu.VMEM),
        scratch_shapes=[
            pltpu.VMEM((N_ROWS, d), jnp.float32),
            pltpu.VMEM((HALF, d), jnp.float32),
            pltpu.VMEM((HALF, d), jnp.bfloat16),
            pltpu.VMEM((HALF, d), jnp.bfloat16),
            pltpu.VMEM((HALF, d), jnp.bfloat16),
            pltpu.SemaphoreType.DMA((2,)),
            pltpu.SemaphoreType.DMA((K,)),
            pltpu.SemaphoreType.DMA((K,)),
            pltpu.SemaphoreType.DMA((K,)),
            pltpu.SemaphoreType.DMA((K,)),
        ],
        compiler_params=pltpu.CompilerParams(collective_id=0),
    )(partial, gamma2)
